# baseline (speedup 1.0000x reference)
import functools

import jax
import jax.numpy as jnp
from jax import lax
from jax.experimental import pallas as pl
from jax.experimental.pallas import tpu as pltpu
from jax.experimental.pallas import tpu_sc as plsc

_NUM_CORES = 2
_NUM_SUBCORES = 16
_NUM_WORKERS = _NUM_CORES * _NUM_SUBCORES

_CHUNK = 8
_NBUF = 2
_TC_SEMS = 8


def _sc_gather(n_tokens, dim):
  n_per_w = n_tokens // _NUM_WORKERS
  n_chunks = n_per_w // _CHUNK

  mesh = plsc.VectorSubcoreMesh(
      core_axis_name="c", subcore_axis_name="s",
      num_cores=_NUM_CORES, num_subcores=_NUM_SUBCORES)

  @functools.partial(
      pl.kernel,
      out_type=jax.ShapeDtypeStruct((n_tokens, dim), jnp.float32),
      mesh=mesh,
      scratch_types=[
          pltpu.VMEM((n_chunks, _CHUNK), jnp.int32),
          pltpu.VMEM((_NBUF, _CHUNK, dim), jnp.float32),
          pltpu.VMEM_SHARED((_NUM_SUBCORES, _NBUF, _CHUNK, dim), jnp.float32),
          [pltpu.SemaphoreType.DMA] * _NBUF,
          [pltpu.SemaphoreType.DMA] * _NBUF,
          [pltpu.SemaphoreType.DMA] * _NBUF,
      ],
  )
  def k(tok_hbm, tab_hbm, out_hbm, idx_v, buf, sp, gsem, xsem, ssem):
    sid = lax.axis_index("s")
    wid = sid * _NUM_CORES + lax.axis_index("c")
    chunk_row = wid * n_chunks
    pltpu.sync_copy(tok_hbm.at[pl.ds(chunk_row, n_chunks)], idx_v)

    def gather_issue(j, b):
      pltpu.async_copy(tab_hbm.at[idx_v.at[j]], buf.at[b], gsem[b])

    def gather_wait(b):
      pltpu.make_async_copy(out_hbm.at[pl.ds(0, _CHUNK)], buf.at[b],
                            gsem[b]).wait()

    def xcopy(b):
      pltpu.async_copy(buf.at[b], sp.at[sid, b], xsem[b])
      pltpu.make_async_copy(buf.at[b], sp.at[sid, b], xsem[b]).wait()

    def store_issue(j, b):
      base = (chunk_row + j) * _CHUNK
      pltpu.async_copy(sp.at[sid, b], out_hbm.at[pl.ds(base, _CHUNK)], ssem[b])

    def store_wait(b):
      pltpu.make_async_copy(sp.at[sid, b], out_hbm.at[pl.ds(0, _CHUNK)],
                            ssem[b]).wait()

    for b in range(_NBUF):
      gather_issue(b, b)

    def body(i, carry):
      for b in range(_NBUF):
        j = _NBUF * i + b
        gather_wait(b)

        @pl.when(j >= _NBUF)
        def _():
          store_wait(b)

        xcopy(b)
        store_issue(j, b)

        @pl.when(j + _NBUF < n_chunks)
        def _():
          gather_issue(j + _NBUF, b)

      return carry

    lax.fori_loop(0, n_chunks // _NBUF, body, 0)

    for b in range(_NBUF):
      store_wait(b)

  return k


def _tc_row_gather(n_tokens, dim):
  def body(tok_smem, tab_hbm, out_hbm, *sems):
    def issue(i8, carry):
      for q in range(_TC_SEMS):
        i = i8 * _TC_SEMS + q
        t = tok_smem[i]
        pltpu.make_async_copy(tab_hbm.at[pl.ds(t, 1)],
                              out_hbm.at[pl.ds(i, 1)], sems[q]).start()
      return carry

    lax.fori_loop(0, n_tokens // _TC_SEMS, issue, 0)

    rows_per_sem = n_tokens // _TC_SEMS
    for q in range(_TC_SEMS):
      pltpu.make_async_copy(tab_hbm.at[pl.ds(0, rows_per_sem)],
                            out_hbm.at[pl.ds(q * rows_per_sem, rows_per_sem)],
                            sems[q]).wait()

  grid_spec = pltpu.PrefetchScalarGridSpec(
      num_scalar_prefetch=1,
      grid=(1,),
      in_specs=[pl.BlockSpec(memory_space=pltpu.MemorySpace.HBM)],
      out_specs=pl.BlockSpec(memory_space=pltpu.MemorySpace.HBM),
      scratch_shapes=[pltpu.SemaphoreType.DMA] * _TC_SEMS,
  )
  return pl.pallas_call(
      body,
      grid_spec=grid_spec,
      out_shape=jax.ShapeDtypeStruct((n_tokens, dim), jnp.float32),
  )


def kernel(t5_tokens, from_table, to_table):
  batch, seq = t5_tokens.shape
  n_tokens = batch * seq
  from_dim = from_table.shape[1]
  to_dim = to_table.shape[1]

  tokens_flat = t5_tokens.reshape(n_tokens)
  tokens2d = t5_tokens.reshape(n_tokens // _CHUNK, _CHUNK)

  out_to = _sc_gather(n_tokens, to_dim)(tokens2d, to_table)
  out_from = _tc_row_gather(n_tokens, from_dim)(tokens_flat, from_table)

  return (out_from.reshape(batch, seq, from_dim),
          out_to.reshape(batch, seq, to_dim))


# speedup vs baseline: 14.1934x; 14.1934x over previous
import functools

import jax
import jax.numpy as jnp
from jax import lax
from jax.experimental import pallas as pl
from jax.experimental.pallas import tpu as pltpu
from jax.experimental.pallas import tpu_sc as plsc

_NUM_CORES = 2
_NUM_SUBCORES = 16
_NUM_WORKERS = _NUM_CORES * _NUM_SUBCORES

_CHUNK = 8
_NBUF = 2


def _dual_gather(n_tokens, from_dim, to_dim):
  n_per_w = n_tokens // _NUM_WORKERS
  n_chunks = n_per_w // _CHUNK

  mesh = plsc.VectorSubcoreMesh(
      core_axis_name="c", subcore_axis_name="s",
      num_cores=_NUM_CORES, num_subcores=_NUM_SUBCORES)

  @functools.partial(
      pl.kernel,
      out_type=(
          jax.ShapeDtypeStruct((n_tokens, from_dim), jnp.float32),
          jax.ShapeDtypeStruct((n_tokens, to_dim), jnp.float32),
      ),
      mesh=mesh,
      scratch_types=[
          pltpu.VMEM((n_chunks, _CHUNK), jnp.int32),
          pltpu.VMEM((_NBUF, _CHUNK, from_dim), jnp.float32),
          pltpu.VMEM((_NBUF, _CHUNK, to_dim), jnp.float32),
          pltpu.VMEM_SHARED((_NUM_SUBCORES, _NBUF, _CHUNK, from_dim),
                            jnp.float32),
          pltpu.VMEM_SHARED((_NUM_SUBCORES, _NBUF, _CHUNK, to_dim),
                            jnp.float32),
          [pltpu.SemaphoreType.DMA] * _NBUF,
          [pltpu.SemaphoreType.DMA] * _NBUF,
          [pltpu.SemaphoreType.DMA] * _NBUF,
          [pltpu.SemaphoreType.DMA] * _NBUF,
          [pltpu.SemaphoreType.DMA] * _NBUF,
          [pltpu.SemaphoreType.DMA] * _NBUF,
      ],
  )
  def k(tok_hbm, from_hbm, to_hbm, out_from_hbm, out_to_hbm,
        idx_v, fbuf, tbuf, fsp, tsp, gf, gt, xf, xt, sf, st):
    sid = lax.axis_index("s")
    wid = sid * _NUM_CORES + lax.axis_index("c")
    chunk_row = wid * n_chunks
    pltpu.sync_copy(tok_hbm.at[pl.ds(chunk_row, n_chunks)], idx_v)

    def gather_issue(j, b):
      pltpu.async_copy(from_hbm.at[idx_v.at[j]], fbuf.at[b], gf[b])
      pltpu.async_copy(to_hbm.at[idx_v.at[j]], tbuf.at[b], gt[b])

    def gather_wait(b):
      pltpu.make_async_copy(out_from_hbm.at[pl.ds(0, _CHUNK)], fbuf.at[b],
                            gf[b]).wait()
      pltpu.make_async_copy(out_to_hbm.at[pl.ds(0, _CHUNK)], tbuf.at[b],
                            gt[b]).wait()

    def xcopy(b):
      pltpu.async_copy(fbuf.at[b], fsp.at[sid, b], xf[b])
      pltpu.async_copy(tbuf.at[b], tsp.at[sid, b], xt[b])
      pltpu.make_async_copy(fbuf.at[b], fsp.at[sid, b], xf[b]).wait()
      pltpu.make_async_copy(tbuf.at[b], tsp.at[sid, b], xt[b]).wait()

    def store_issue(j, b):
      base = (chunk_row + j) * _CHUNK
      pltpu.async_copy(fsp.at[sid, b], out_from_hbm.at[pl.ds(base, _CHUNK)],
                       sf[b])
      pltpu.async_copy(tsp.at[sid, b], out_to_hbm.at[pl.ds(base, _CHUNK)],
                       st[b])

    def store_wait(b):
      pltpu.make_async_copy(fsp.at[sid, b], out_from_hbm.at[pl.ds(0, _CHUNK)],
                            sf[b]).wait()
      pltpu.make_async_copy(tsp.at[sid, b], out_to_hbm.at[pl.ds(0, _CHUNK)],
                            st[b]).wait()

    for b in range(_NBUF):
      gather_issue(b, b)

    def body(i, carry):
      for b in range(_NBUF):
        j = _NBUF * i + b
        gather_wait(b)

        @pl.when(j >= _NBUF)
        def _():
          store_wait(b)

        xcopy(b)
        store_issue(j, b)

        @pl.when(j + _NBUF < n_chunks)
        def _():
          gather_issue(j + _NBUF, b)

      return carry

    lax.fori_loop(0, n_chunks // _NBUF, body, 0)

    for b in range(_NBUF):
      store_wait(b)

  return k


def kernel(t5_tokens, from_table, to_table):
  batch, seq = t5_tokens.shape
  n_tokens = batch * seq
  from_dim = from_table.shape[1]
  to_dim = to_table.shape[1]

  tokens2d = t5_tokens.reshape(n_tokens // _CHUNK, _CHUNK)
  gather = _dual_gather(n_tokens, from_dim, to_dim)
  out_from, out_to = gather(tokens2d, from_table, to_table)
  return (out_from.reshape(batch, seq, from_dim),
          out_to.reshape(batch, seq, to_dim))
